# baseline (speedup 1.0000x reference)
import functools

import jax
import jax.numpy as jnp
from jax import lax
from jax.experimental import pallas as pl
from jax.experimental.pallas import tpu as pltpu
from jax.experimental.pallas import tpu_sc as plsc

EMB = 128
BATCH = 16384
NUM_TABLES = 5
TOTAL_ROWS = BATCH * NUM_TABLES

_info = plsc.get_sparse_core_info()
NC, NS = _info.num_cores, _info.num_subcores
NW = NC * NS
ROWS_PER_W = TOTAL_ROWS // NW
G = 64
STEPS = ROWS_PER_W // G
NBUF = 6
NBLK = BATCH // G

BASES = (0, 6048, 6056, 6064, 6088)
TAB_ROWS = 9528

_STAGE = (
    (0, 0, 1512, 0),
    (0, 1512, 1512, 1512),
    (0, 3024, 1512, 3024),
    (0, 4536, 1505, 4536),
    (1, 0, 2, 6048),
    (2, 0, 7, 6056),
    (3, 0, 21, 6064),
    (4, 0, 1144, 6088),
    (4, 1144, 1144, 7232),
    (4, 2288, 1151, 8376),
)


@functools.partial(
    pl.kernel,
    mesh=plsc.VectorSubcoreMesh(core_axis_name="c", subcore_axis_name="s"),
    out_type=jax.ShapeDtypeStruct((BATCH, NUM_TABLES * EMB), jnp.float32),
    scratch_types=[
        pltpu.VMEM((1, STEPS, G), jnp.int32),
        pltpu.VMEM((NBUF, G, EMB), jnp.float32),
        pltpu.VMEM_SHARED((TAB_ROWS, EMB), jnp.float32),
    ]
    + [pltpu.SemaphoreType.DMA] * (2 * NBUF),
)
def _gather_kernel(idx_hbm, user_hbm, gender_hbm, age_hbm, occup_hbm,
                   zip_hbm, out_hbm, idx_v, bufs_v, shared_v, *sems):
    tabs = (user_hbm, gender_hbm, age_hbm, occup_hbm, zip_hbm)
    sem_g = sems[:NBUF]
    sem_w = sems[NBUF:]
    sid = lax.axis_index("s")
    wid = sid * NC + lax.axis_index("c")

    h_idx = pltpu.async_copy(idx_hbm.at[pl.ds(wid, 1)], idx_v, sems[0])
    for t, (arg, soff, n, doff) in enumerate(_STAGE):
        @pl.when(sid == t)
        def _stage(arg=arg, soff=soff, n=n, doff=doff):
            pltpu.sync_copy(tabs[arg].at[pl.ds(soff, n)],
                            shared_v.at[pl.ds(doff, n)])

    h_idx.wait()
    plsc.subcore_barrier()

    hg = {}
    hw = {}

    def fire_write(w):
        u = wid * STEPS + w
        j = lax.shift_right_logical(u, NBLK.bit_length() - 1)
        k = jnp.bitwise_and(u, NBLK - 1)
        row = pl.multiple_of(k * G, G)
        col = pl.multiple_of(j * EMB, EMB)
        hg[w].wait()
        hw[w] = pltpu.async_copy(
            bufs_v.at[w % NBUF],
            out_hbm.at[pl.ds(row, G), pl.ds(col, EMB)],
            sem_w[w % NBUF])

    for s in range(STEPS):
        b = s % NBUF
        if s >= NBUF:
            hw[s - NBUF].wait()
        hg[s] = pltpu.async_copy(shared_v.at[idx_v.at[0, s]], bufs_v.at[b],
                                 sem_g[b])
        if s >= NBUF - 1:
            fire_write(s - (NBUF - 1))
    for w in range(STEPS - (NBUF - 1), STEPS):
        fire_write(w)
    for w in range(STEPS - NBUF, STEPS):
        hw[w].wait()


def kernel(categorical_feats, user_table, gender_table, age_table,
           occup_table, zip_table):
    offs = jnp.array([BASES[0] - 1] + list(BASES[1:]), dtype=jnp.int32)
    idx = (categorical_feats.astype(jnp.int32) + offs[None, :]).T.reshape(
        NW, STEPS, G)
    return _gather_kernel(idx, user_table, gender_table, age_table,
                          occup_table, zip_table)
